# baseline (speedup 1.0000x reference)
import functools

import jax
import jax.numpy as jnp
from jax import lax
from jax.experimental import pallas as pl
from jax.experimental.pallas import tpu as pltpu
from jax.experimental.pallas import tpu_sc as plsc

_T = 512
_ROWS_PER_W = 5
_CHUNKS = _T // 16
_THD = 0.7
_THD_INV = 1.0 / 0.7


def _sc_partials(p_flat, l_flat):
    info = plsc.get_sparse_core_info()
    NC, NS = info.num_cores, info.num_subcores
    NW = NC * NS
    elems = _ROWS_PER_W * _T
    mesh = plsc.VectorSubcoreMesh(core_axis_name="c", subcore_axis_name="s")

    @functools.partial(
        pl.kernel,
        mesh=mesh,
        compiler_params=pltpu.CompilerParams(needs_layout_passes=False),
        out_type=jax.ShapeDtypeStruct((NW, 16), jnp.float32),
        scratch_types=[
            pltpu.VMEM((elems,), jnp.float32),
            pltpu.VMEM((elems,), jnp.float32),
            pltpu.VMEM((_T,), jnp.float32),
            pltpu.VMEM((16,), jnp.float32),
        ],
    )
    def k(p_hbm, l_hbm, out_hbm, p_v, l_v, all_v, tmp_v):
        wid = lax.axis_index("s") * NC + lax.axis_index("c")
        base = wid * elems
        pltpu.sync_copy(p_hbm.at[pl.ds(base, elems)], p_v)
        pltpu.sync_copy(l_hbm.at[pl.ds(base, elems)], l_v)
        iota = lax.iota(jnp.int32, 16)
        zero16 = jnp.zeros((16,), jnp.float32)
        tp_vec = zero16
        cnt_vec = zero16
        for r in range(_ROWS_PER_W):
            def chunk_body(i, carry, r=r):
                P_c, L_c, SsP_c, SsL_c, tp_vec, cnt_vec = carry
                idx = r * _T + i * 16 + iota
                p = plsc.load_gather(p_v, [idx])
                l = plsc.load_gather(l_v, [idx])
                a = jnp.maximum(p, l)
                plsc.store_scatter(all_v, [i * 16 + iota], a)
                im1 = i * 16 - 1 + iota
                prevg = plsc.load_gather(all_v, [jnp.maximum(im1, 0)])
                prev = jnp.where(im1 >= 0, prevg, 0.0)
                cP = plsc.cumsum(p) + P_c
                cL = plsc.cumsum(l) + L_c
                is_start = a * (1.0 - prev)
                mP = jnp.where(is_start > 0, cP - p, -1.0)
                mL = jnp.where(is_start > 0, cL - l, -1.0)
                SsP = jnp.maximum(plsc.cummax(mP), SsP_c)
                SsL = jnp.maximum(plsc.cummax(mL), SsL_c)
                is_endb = prev * (1.0 - a)
                ratio = (cP - SsP) / ((cL - SsL) + 1e-7)
                in_rng = jnp.logical_and(ratio >= _THD, ratio < _THD_INV)
                tp_vec = tp_vec + jnp.where(
                    jnp.logical_and(is_endb > 0, in_rng), 1.0, 0.0)
                cnt_vec = cnt_vec + is_start
                P_n = jnp.broadcast_to(jnp.max(cP), (16,))
                L_n = jnp.broadcast_to(jnp.max(cL), (16,))
                SsP_n = jnp.broadcast_to(jnp.max(SsP), (16,))
                SsL_n = jnp.broadcast_to(jnp.max(SsL), (16,))
                return (P_n, L_n, SsP_n, SsL_n, tp_vec, cnt_vec)

            init = (zero16, zero16, zero16 - 1.0, zero16 - 1.0, tp_vec, cnt_vec)
            P_f, L_f, SsP_f, SsL_f, tp_vec, cnt_vec = lax.fori_loop(
                0, _CHUNKS, chunk_body, init)
            fa = plsc.load_gather(all_v, [jnp.full((16,), _T - 1, jnp.int32)])
            rf = (P_f - SsP_f) / ((L_f - SsL_f) + 1e-7)
            in_f = jnp.logical_and(rf >= _THD, rf < _THD_INV)
            tp_vec = tp_vec + jnp.where(
                jnp.logical_and(jnp.logical_and(fa > 0, in_f), iota == 0),
                1.0, 0.0)
        tp_tot = jnp.sum(tp_vec)
        cnt_tot = jnp.sum(cnt_vec)
        outv = jnp.where(iota == 0, tp_tot, jnp.where(iota == 1, cnt_tot, 0.0))
        tmp_v[...] = outv
        pltpu.sync_copy(tmp_v, out_hbm.at[wid])

    return k(p_flat, l_flat)


def _combine_kernel(x_ref, o_ref):
    x = x_ref[...]
    pair = x.reshape(16, 2, 16).sum(axis=1)
    col = lax.broadcasted_iota(jnp.int32, (16, 16), 1)
    tp = jnp.sum(jnp.where(col == 0, pair, 0.0), axis=1, keepdims=True)
    cnt = jnp.sum(jnp.where(col == 1, pair, 0.0), axis=1, keepdims=True)
    denom = 0.5 * tp + 0.5 * cnt
    f = jnp.where(denom > 0, tp / denom, 0.0)
    o_ref[...] = jnp.sum(f, axis=(0, 1), keepdims=True) / 16


@jax.jit
def kernel(strong_preds, ground_truths):
    p = strong_preds.reshape(-1)
    l = ground_truths.reshape(-1)
    partial = _sc_partials(p, l)
    out = pl.pallas_call(
        _combine_kernel,
        out_shape=jax.ShapeDtypeStruct((1, 1), jnp.float32),
    )(partial)
    return out[0, 0]
